# baseline (speedup 1.0000x reference)
import jax
import jax.numpy as jnp
from jax.experimental import pallas as pl

_TOPK = 8
_NUM_EXPERTS = 64
_BLOCK_ROWS = 512


def _top8(scores_t):
    iota = jax.lax.broadcasted_iota(jnp.int32, scores_t.shape, 0)
    vals = scores_t
    neg_inf = jnp.float32(-jnp.inf)
    top_v = []
    top_i = []
    for _ in range(_TOPK):
        m = jnp.max(vals, axis=0, keepdims=True)
        idx = jnp.min(jnp.where(vals == m, iota, _NUM_EXPERTS),
                      axis=0, keepdims=True)
        top_v.append(m)
        top_i.append(idx)
        vals = jnp.where(iota == idx, neg_inf, vals)
    v = jnp.concatenate(top_v, axis=0)
    i = jnp.concatenate(top_i, axis=0)
    return (v / jnp.sum(v, axis=0, keepdims=True)).T, i.T


def _gate_kernel(xa_ref, xb_ref, e_ref, b_ref, w_ref, i_ref, s_ref):
    e = e_ref[...]
    b = b_ref[...]
    for h, x_blk in ((0, xa_ref[...]), (1, xb_ref[...])):
        logits_t = jax.lax.dot_general(
            e, x_blk,
            dimension_numbers=(((1,), (1,)), ((), ())),
            preferred_element_type=jnp.float32,
        )
        scores_t = jax.nn.sigmoid(logits_t) + b
        s_ref[h] = scores_t.T
        w, i = _top8(scores_t)
        w_ref[h] = w
        i_ref[h] = i


@jax.jit
def kernel(x, expert_embeddings, bias):
    n_rows, n_cols = x.shape
    n_exp = expert_embeddings.shape[0]
    half_blocks = n_rows // (2 * _BLOCK_ROWS)
    half = n_rows // 2
    grid = (half_blocks,)
    bias2d = bias.reshape(n_exp, 1)
    weights, indices, scores = pl.pallas_call(
        _gate_kernel,
        grid=grid,
        in_specs=[
            pl.BlockSpec((_BLOCK_ROWS, n_cols), lambda i: (i, 0)),
            pl.BlockSpec((_BLOCK_ROWS, n_cols),
                         lambda i, hb=half_blocks: (i + hb, 0)),
            pl.BlockSpec((n_exp, n_cols), lambda i: (0, 0)),
            pl.BlockSpec((n_exp, 1), lambda i: (0, 0)),
        ],
        out_specs=[
            pl.BlockSpec((2, _BLOCK_ROWS, _TOPK), lambda i: (0, i, 0)),
            pl.BlockSpec((2, _BLOCK_ROWS, _TOPK), lambda i: (0, i, 0)),
            pl.BlockSpec((2, _BLOCK_ROWS, n_exp), lambda i: (0, i, 0)),
        ],
        out_shape=[
            jax.ShapeDtypeStruct((2, half, _TOPK), jnp.float32),
            jax.ShapeDtypeStruct((2, half, _TOPK), jnp.int32),
            jax.ShapeDtypeStruct((2, half, n_exp), jnp.float32),
        ],
    )(x, x, expert_embeddings, bias2d)
    return (weights.reshape(n_rows, _TOPK).astype(x.dtype),
            indices.reshape(n_rows, _TOPK),
            scores.reshape(n_rows, n_exp))
